# baseline (speedup 1.0000x reference)
import functools

import jax
import jax.numpy as jnp
from jax.experimental import pallas as pl
from jax.experimental.pallas import tpu as pltpu

NUM_EMBEDDINGS = 8192
EMBEDDING_DIM = 32
LATENT_DIM = 256
B_TOTAL = 16384
BB = 256


def _vq_block_kernel(z_ref, wi_ref, bi_ref, cb_ref, wo_ref, bo_ref,
                     out_ref, idx_ref, ze_ref, sq_ref):
    i = pl.program_id(0)

    ze = jax.lax.dot_general(
        z_ref[...], wi_ref[...], (((1,), (0,)), ((), ())),
        preferred_element_type=jnp.float32) + bi_ref[...]

    ze_norm = jnp.sqrt(jnp.sum(ze * ze, axis=1, keepdims=True))
    ze_n = ze / jnp.maximum(ze_norm, 1e-12)
    w = cb_ref[...]
    w_norm = jnp.sqrt(jnp.sum(w * w, axis=1, keepdims=True))
    w_n = w / jnp.maximum(w_norm, 1e-12)

    s = jax.lax.dot_general(
        ze_n, w_n, (((1,), (1,)), ((), ())),
        preferred_element_type=jnp.float32)
    row2 = jnp.sum(ze_n * ze_n, axis=1, keepdims=True)
    wn2 = jnp.sum(w_n * w_n, axis=1).reshape(1, NUM_EMBEDDINGS)
    d = row2 - 2.0 * s + wn2

    H = NUM_EMBEDDINGS // 2
    iota = jax.lax.broadcasted_iota(jnp.int32, (BB, H), 1)
    d0, d1 = d[:, :H], d[:, H:]
    m0 = jnp.min(d0, axis=1, keepdims=True)
    i0 = jnp.min(jnp.where(d0 == m0, iota, NUM_EMBEDDINGS), axis=1)
    m1 = jnp.min(d1, axis=1, keepdims=True)
    i1 = jnp.min(jnp.where(d1 == m1, iota + H, NUM_EMBEDDINGS), axis=1)
    u = pltpu.bitcast(m0, jnp.uint32)
    u = (u + jnp.uint32(0x7FFF) + ((u >> 16) & jnp.uint32(1))) & jnp.uint32(0xFFFF0000)
    m0r = pltpu.bitcast(u, jnp.float32)
    take1 = (m1 < m0r).reshape(BB)
    idx = jnp.where(take1, i1, i0)
    idx_ref[...] = idx.reshape(1, 1, BB)

    iota_full = jax.lax.broadcasted_iota(jnp.int32, (BB, NUM_EMBEDDINGS), 1)
    oh = (iota_full == idx[:, None]).astype(jnp.float32)
    z_q = jax.lax.dot_general(
        oh, w, (((1,), (0,)), ((), ())),
        preferred_element_type=jnp.float32,
        precision=jax.lax.Precision.HIGHEST)

    part = jnp.sum((ze - z_q) ** 2).reshape(1, 1)

    @pl.when(i == 0)
    def _():
        sq_ref[...] = jnp.zeros_like(sq_ref)

    sq_ref[...] += part

    out_ref[...] = jax.lax.dot_general(
        z_q, wo_ref[...], (((1,), (0,)), ((), ())),
        preferred_element_type=jnp.float32) + bo_ref[...]

    ze_ref[...] = ze


@jax.jit
def kernel(z_e, W_in, b_in, codebook, W_out, b_out):
    nb = B_TOTAL // BB
    out, idx3, ze, sq = pl.pallas_call(
        _vq_block_kernel,
        grid=(nb,),
        in_specs=[
            pl.BlockSpec((BB, LATENT_DIM), lambda i: (i, 0)),
            pl.BlockSpec((LATENT_DIM, EMBEDDING_DIM), lambda i: (0, 0)),
            pl.BlockSpec((1, EMBEDDING_DIM), lambda i: (0, 0)),
            pl.BlockSpec((NUM_EMBEDDINGS, EMBEDDING_DIM), lambda i: (0, 0)),
            pl.BlockSpec((EMBEDDING_DIM, LATENT_DIM), lambda i: (0, 0)),
            pl.BlockSpec((1, LATENT_DIM), lambda i: (0, 0)),
        ],
        out_specs=[
            pl.BlockSpec((BB, LATENT_DIM), lambda i: (i, 0)),
            pl.BlockSpec((1, 1, BB), lambda i: (i, 0, 0)),
            pl.BlockSpec((BB, EMBEDDING_DIM), lambda i: (i, 0)),
            pl.BlockSpec((1, 1), lambda i: (0, 0)),
        ],
        out_shape=[
            jax.ShapeDtypeStruct((B_TOTAL, LATENT_DIM), jnp.float32),
            jax.ShapeDtypeStruct((nb, 1, BB), jnp.int32),
            jax.ShapeDtypeStruct((B_TOTAL, EMBEDDING_DIM), jnp.float32),
            jax.ShapeDtypeStruct((1, 1), jnp.float32),
        ],
    )(z_e, W_in, b_in.reshape(1, EMBEDDING_DIM), codebook,
      W_out, b_out.reshape(1, LATENT_DIM))

    encoding_indices = idx3.reshape(B_TOTAL)
    encoder_loss = (sq[0, 0] / (B_TOTAL * EMBEDDING_DIM)).reshape(())
    codebook_loss = encoder_loss
    loss = encoder_loss * COMMITMENT_COST_
    return (out, loss, encoder_loss, codebook_loss, encoding_indices, ze)


COMMITMENT_COST_ = 0.25
